# baseline (speedup 1.0000x reference)
import functools

import jax
import jax.numpy as jnp
from jax import lax
from jax.experimental import pallas as pl
from jax.experimental.pallas import tpu as pltpu
from jax.experimental.pallas import tpu_sc as plsc

B = 16
S = 2048
D = 768
L = 16
NC = 2
NS = 16
NW = NC * NS
NLANE = D // L
CHUNK = 16


def _span_mean_body(sent_hbm, starts_hbm, ends_hbm, out_hbm,
                    idx_ref, rows_ref, sref, eref, res_ref, sem):
    w = lax.axis_index("s") * NC + lax.axis_index("c")

    pltpu.sync_copy(starts_hbm, sref)
    pltpu.sync_copy(ends_hbm, eref)
    lanes = lax.broadcasted_iota(jnp.int32, (L,), 0)
    wvec = jnp.full((L,), w, jnp.int32)
    start_b = plsc.load_gather(sref, [wvec])
    end_b = plsc.load_gather(eref, [wvec])
    limit = end_b - 1

    zero = jnp.zeros((L,), jnp.float32)
    carry_init = (jnp.int32(0),) + (zero,) * NLANE

    def cond(carry):
        k = carry[0]
        return jnp.any(start_b + k * CHUNK < end_b)

    def body(carry):
        k, acc = carry[0], carry[1:]
        idx_ref[...] = jnp.minimum(start_b + k * CHUNK + lanes, limit)
        pltpu.async_copy(sent_hbm.at[idx_ref], rows_ref, sem).wait()

        def row_body(r, a):
            return tuple(
                a[j] + rows_ref[r, pl.ds(j * L, L)] for j in range(NLANE)
            )

        return (k + 1,) + lax.fori_loop(0, CHUNK, row_body, acc)

    carry = lax.while_loop(cond, body, carry_init)
    k_final, acc = carry[0], carry[1:]

    n_f = (end_b - start_b).astype(jnp.float32)
    extra = (k_final * CHUNK).astype(jnp.float32) - n_f
    inv_n = 1.0 / n_f
    for j in range(NLANE):
        last = rows_ref[CHUNK - 1, pl.ds(j * L, L)]
        res_ref[0, pl.ds(j * L, L)] = (acc[j] - extra * last) * inv_n

    pltpu.sync_copy(res_ref, out_hbm.at[pl.ds(w, 1)])


_span_mean = functools.partial(
    pl.kernel,
    out_type=jax.ShapeDtypeStruct((NW, D), jnp.float32),
    mesh=plsc.VectorSubcoreMesh(core_axis_name="c", subcore_axis_name="s",
                                num_cores=NC, num_subcores=NS),
    compiler_params=pltpu.CompilerParams(needs_layout_passes=False),
    scratch_types=[
        pltpu.VMEM((CHUNK,), jnp.int32),
        pltpu.VMEM((CHUNK, D), jnp.float32),
        pltpu.VMEM((NW,), jnp.int32),
        pltpu.VMEM((NW,), jnp.int32),
        pltpu.VMEM((1, D), jnp.float32),
        pltpu.SemaphoreType.DMA,
    ],
)(_span_mean_body)


@jax.jit
def kernel(sent, positions):
    pos = positions.astype(jnp.int32)
    base = jnp.arange(B, dtype=jnp.int32) * S
    starts = jnp.concatenate([base + pos[:, 0], base + pos[:, 2]])
    ends = jnp.concatenate([base + pos[:, 1], base + pos[:, 3]])
    out = _span_mean(sent.reshape(B * S, D), starts, ends)
    return out[:B], out[B:]


# speedup vs baseline: 1.7249x; 1.7249x over previous
import functools

import jax
import jax.numpy as jnp
from jax import lax
from jax.experimental import pallas as pl
from jax.experimental.pallas import tpu as pltpu
from jax.experimental.pallas import tpu_sc as plsc

B = 16
S = 2048
D = 768
L = 16
NC = 2
NS = 16
NW = NC * NS
NLANE = D // L
CHUNK = 32


def _span_mean_body(sent_hbm, starts_hbm, ends_hbm, out_hbm,
                    idx0_ref, idx1_ref, rows0_ref, rows1_ref,
                    sref, eref, res_ref, sem0, sem1):
    w = lax.axis_index("s") * NC + lax.axis_index("c")

    pltpu.sync_copy(starts_hbm, sref)
    pltpu.sync_copy(ends_hbm, eref)
    lanes = lax.broadcasted_iota(jnp.int32, (L,), 0)
    wvec = jnp.full((L,), w, jnp.int32)
    start_b = plsc.load_gather(sref, [wvec])
    end_b = plsc.load_gather(eref, [wvec])
    limit = end_b - 1

    def fill_idx(idx_ref, k):
        base = start_b + k * CHUNK
        for u in range(CHUNK // L):
            idx_ref[pl.ds(u * L, L)] = jnp.minimum(base + u * L + lanes, limit)

    def issue(idx_ref, rows_ref, sem, k):
        fill_idx(idx_ref, k)
        pltpu.async_copy(sent_hbm.at[idx_ref], rows_ref, sem)

    def consume(rows_ref, sem, acc):
        pltpu.make_async_copy(sent_hbm.at[idx0_ref], rows_ref, sem).wait()

        def row_body(r, a):
            return tuple(
                a[j] + rows_ref[r, pl.ds(j * L, L)] for j in range(NLANE)
            )

        return lax.fori_loop(0, CHUNK, row_body, acc)

    zero = jnp.zeros((L,), jnp.float32)
    carry_init = (jnp.int32(0),) + (zero,) * NLANE

    issue(idx0_ref, rows0_ref, sem0, 0)

    def cond(carry):
        k2 = carry[0]
        return jnp.any(start_b + k2 * (2 * CHUNK) < end_b)

    def body(carry):
        k2, acc = carry[0], carry[1:]
        issue(idx1_ref, rows1_ref, sem1, 2 * k2 + 1)
        acc = consume(rows0_ref, sem0, acc)
        issue(idx0_ref, rows0_ref, sem0, 2 * k2 + 2)
        acc = consume(rows1_ref, sem1, acc)
        return (k2 + 1,) + acc

    carry = lax.while_loop(cond, body, carry_init)
    k2_final, acc = carry[0], carry[1:]
    pltpu.make_async_copy(sent_hbm.at[idx0_ref], rows0_ref, sem0).wait()

    n_f = (end_b - start_b).astype(jnp.float32)
    extra = (k2_final * (2 * CHUNK)).astype(jnp.float32) - n_f
    inv_n = 1.0 / n_f
    for j in range(NLANE):
        last = rows1_ref[CHUNK - 1, pl.ds(j * L, L)]
        res_ref[0, pl.ds(j * L, L)] = (acc[j] - extra * last) * inv_n

    pltpu.sync_copy(res_ref, out_hbm.at[pl.ds(w, 1)])


_span_mean = functools.partial(
    pl.kernel,
    out_type=jax.ShapeDtypeStruct((NW, D), jnp.float32),
    mesh=plsc.VectorSubcoreMesh(core_axis_name="c", subcore_axis_name="s",
                                num_cores=NC, num_subcores=NS),
    compiler_params=pltpu.CompilerParams(needs_layout_passes=False),
    scratch_types=[
        pltpu.VMEM((CHUNK,), jnp.int32),
        pltpu.VMEM((CHUNK,), jnp.int32),
        pltpu.VMEM((CHUNK, D), jnp.float32),
        pltpu.VMEM((CHUNK, D), jnp.float32),
        pltpu.VMEM((NW,), jnp.int32),
        pltpu.VMEM((NW,), jnp.int32),
        pltpu.VMEM((1, D), jnp.float32),
        pltpu.SemaphoreType.DMA,
        pltpu.SemaphoreType.DMA,
    ],
)(_span_mean_body)


@jax.jit
def kernel(sent, positions):
    pos = positions.astype(jnp.int32)
    base = jnp.arange(B, dtype=jnp.int32) * S
    starts = jnp.concatenate([base + pos[:, 0], base + pos[:, 2]])
    ends = jnp.concatenate([base + pos[:, 1], base + pos[:, 3]])
    out = _span_mean(sent.reshape(B * S, D), starts, ends)
    return out[:B], out[B:]


# speedup vs baseline: 1.9038x; 1.1037x over previous
import functools

import jax
import jax.numpy as jnp
from jax import lax
from jax.experimental import pallas as pl
from jax.experimental.pallas import tpu as pltpu
from jax.experimental.pallas import tpu_sc as plsc

B = 16
S = 2048
D = 768
L = 16
NC = 2
NS = 16
NW = NC * NS
NLANE = D // L
CHUNK = 32
PAD = 8
MAXBASE = B * S - (CHUNK + PAD)


def _span_mean_body(sent_hbm, starts_hbm, ends_hbm, out_hbm,
                    rows0_ref, rows1_ref, sref, eref, res_ref, sem0, sem1):
    c = lax.axis_index("c")
    s = lax.axis_index("s")
    w = c * NS + s

    pltpu.sync_copy(starts_hbm, sref)
    pltpu.sync_copy(ends_hbm, eref)
    lanes = lax.broadcasted_iota(jnp.int32, (L,), 0)
    onehot = lanes == s
    svec = jnp.where(c == 0, sref[0:L], sref[L:2 * L])
    evec = jnp.where(c == 0, eref[0:L], eref[L:2 * L])
    start = jnp.max(jnp.where(onehot, svec, 0))
    end = jnp.max(jnp.where(onehot, evec, 0))
    n = end - start

    def aligned_base(k):
        base = start + k * CHUNK
        abase = jnp.minimum((base // PAD) * PAD, MAXBASE)
        return pl.multiple_of(abase, PAD), base - abase

    def issue(rows_ref, sem, k):
        abase, _ = aligned_base(k)
        pltpu.async_copy(sent_hbm.at[pl.ds(abase, CHUNK + PAD)], rows_ref, sem)

    def consume(rows_ref, sem, k, acc):
        pltpu.make_async_copy(
            sent_hbm.at[pl.ds(0, CHUNK + PAD)], rows_ref, sem).wait()
        _, delta = aligned_base(k)
        cnt = jnp.minimum(n - k * CHUNK, CHUNK)

        def row_body(r, a):
            return tuple(
                a[j] + rows_ref[delta + r, pl.ds(j * L, L)]
                for j in range(NLANE)
            )

        return lax.fori_loop(0, cnt, row_body, acc)

    zero = jnp.zeros((L,), jnp.float32)
    acc_init = (zero,) * NLANE
    nchunks = (n + CHUNK - 1) // CHUNK
    npairs = (nchunks + 1) // 2

    issue(rows0_ref, sem0, 0)

    def body(k2, acc):
        issue(rows1_ref, sem1, 2 * k2 + 1)
        acc = consume(rows0_ref, sem0, 2 * k2, acc)
        issue(rows0_ref, sem0, 2 * k2 + 2)
        acc = consume(rows1_ref, sem1, 2 * k2 + 1, acc)
        return acc

    acc = lax.fori_loop(0, npairs, body, acc_init)
    pltpu.make_async_copy(
        sent_hbm.at[pl.ds(0, CHUNK + PAD)], rows0_ref, sem0).wait()

    inv_n = 1.0 / jnp.full((L,), n, jnp.float32)
    for j in range(NLANE):
        res_ref[0, pl.ds(j * L, L)] = acc[j] * inv_n

    pltpu.sync_copy(res_ref, out_hbm.at[pl.ds(w, 1)])


_span_mean = functools.partial(
    pl.kernel,
    out_type=jax.ShapeDtypeStruct((NW, D), jnp.float32),
    mesh=plsc.VectorSubcoreMesh(core_axis_name="c", subcore_axis_name="s",
                                num_cores=NC, num_subcores=NS),
    compiler_params=pltpu.CompilerParams(needs_layout_passes=False),
    scratch_types=[
        pltpu.VMEM((CHUNK + PAD, D), jnp.float32),
        pltpu.VMEM((CHUNK + PAD, D), jnp.float32),
        pltpu.VMEM((NW,), jnp.int32),
        pltpu.VMEM((NW,), jnp.int32),
        pltpu.VMEM((1, D), jnp.float32),
        pltpu.SemaphoreType.DMA,
        pltpu.SemaphoreType.DMA,
    ],
)(_span_mean_body)


@jax.jit
def kernel(sent, positions):
    pos = positions.astype(jnp.int32)
    base = jnp.arange(B, dtype=jnp.int32) * S
    starts = jnp.concatenate([base + pos[:, 0], base + pos[:, 2]])
    ends = jnp.concatenate([base + pos[:, 1], base + pos[:, 3]])
    out = _span_mean(sent.reshape(B * S, D), starts, ends)
    return out[:B], out[B:]
